# baseline (speedup 1.0000x reference)
import functools

import jax
import jax.numpy as jnp
from jax import lax
from jax.experimental import pallas as pl
from jax.experimental.pallas import tpu as pltpu
from jax.experimental.pallas import tpu_sc as plsc

V, D, S, K, M = 100000, 128, 2048, 8, 8192
NC, NS, L = 2, 16, 16
NW = NC * NS
ROWS_A = S // NW
WPK = NW // K
KPC = K // NC
RPW = M // WPK
CHUNK = 64
NBUF = 8
NCHUNK = RPW // CHUNK
GPC = CHUNK // L
DL = D // L
TRASH = S

_mesh = plsc.VectorSubcoreMesh(core_axis_name="c", subcore_axis_name="s")


def _bcast(x, lane):
    idx = jnp.broadcast_to(jnp.asarray(lane, jnp.int32), (L,))
    return jnp.take_along_axis(x, idx, axis=0, mode="promise_in_bounds")


def _lane_reduce(x, op):
    lanes = lax.iota(jnp.int32, L)
    for sh in (8, 4, 2, 1):
        idx = jnp.bitwise_xor(lanes, sh)
        x = op(x, jnp.take_along_axis(x, idx, axis=0,
                                      mode="promise_in_bounds"))
    return x


@functools.partial(
    pl.kernel,
    out_type=jax.ShapeDtypeStruct((S + 8, D), jnp.float32),
    mesh=_mesh,
    scratch_types=[
        pltpu.VMEM((ROWS_A,), jnp.int32),
        pltpu.VMEM((ROWS_A,), jnp.int32),
        pltpu.VMEM((ROWS_A, D), jnp.float32),
        pltpu.VMEM((M,), jnp.float32),
        pltpu.VMEM((RPW,), jnp.int32),
        pltpu.VMEM((RPW,), jnp.float32),
        pltpu.VMEM((L,), jnp.int32),
        pltpu.VMEM((CHUNK, D), jnp.float32),
        pltpu.VMEM((CHUNK, D), jnp.float32),
        pltpu.VMEM((CHUNK, D), jnp.float32),
        pltpu.VMEM((CHUNK, D), jnp.float32),
        pltpu.VMEM((CHUNK, D), jnp.float32),
        pltpu.VMEM((CHUNK, D), jnp.float32),
        pltpu.VMEM((CHUNK, D), jnp.float32),
        pltpu.VMEM((CHUNK, D), jnp.float32),
        pltpu.VMEM((NS, D), jnp.float32),
        pltpu.VMEM((L, D), jnp.float32),
        pltpu.VMEM((L,), jnp.int32),
        pltpu.VMEM_SHARED((NS, D), jnp.float32),
        pltpu.SemaphoreType.DMA,
        pltpu.SemaphoreType.DMA,
        pltpu.SemaphoreType.DMA,
        pltpu.SemaphoreType.DMA,
        pltpu.SemaphoreType.DMA,
        pltpu.SemaphoreType.DMA,
        pltpu.SemaphoreType.DMA,
        pltpu.SemaphoreType.DMA,
        pltpu.SemaphoreType.DMA,
        pltpu.SemaphoreType.DMA,
        pltpu.SemaphoreType.DMA,
        pltpu.SemaphoreType.DMA,
        pltpu.SemaphoreType.DMA,
        pltpu.SemaphoreType.DMA,
    ],
)
def _sc_combiner(table, src_idx, pos16, masks, weights,
                 out, sidx, didx, srows, wbuf, midx_all, cbuf, posv,
                 buf0, buf1, buf2, buf3, buf4, buf5, buf6, buf7,
                 comb, crow, didx16, shared,
                 sem_w, sem_m, sem_s, sem_p, sem_g, sem_o,
                 sem_b0, sem_b1, sem_b2, sem_b3,
                 sem_b4, sem_b5, sem_b6, sem_b7):
    c = lax.axis_index("c")
    s = lax.axis_index("s")
    wid = c * NS + s
    k = c * KPC + s // WPK
    q = s % WPK
    base = wid * ROWS_A

    cp_w = pltpu.async_copy(weights.at[k], wbuf, sem_w)
    cp_m = pltpu.async_copy(masks.at[k, pl.ds(q * RPW, RPW)], midx_all, sem_m)
    cp_si = pltpu.async_copy(src_idx.at[pl.ds(base, ROWS_A)], sidx, sem_s)
    cp_p = pltpu.async_copy(pos16, posv.at[pl.ds(0, K)], sem_p)

    cp_si.wait()
    cp_sr = pltpu.async_copy(table.at[sidx], srows, sem_g)

    cp_m.wait()
    bufs = [buf0, buf1, buf2, buf3, buf4, buf5, buf6, buf7]
    sems = [sem_b0, sem_b1, sem_b2, sem_b3, sem_b4, sem_b5, sem_b6, sem_b7]
    handles = [
        pltpu.async_copy(table.at[midx_all.at[pl.ds(t * CHUNK, CHUNK)]],
                         bufs[t], sems[t])
        for t in range(NBUF)
    ]

    cp_p.wait()
    lanes = lax.iota(jnp.int32, L)
    pos = jnp.where(lanes < K, posv[...], TRASH)
    nxt = jnp.take_along_axis(pos, jnp.minimum(lanes + 1, L - 1), axis=0,
                              mode="promise_in_bounds")
    winner = (lanes < K) & ((pos != nxt) | (lanes == K - 1))
    dstp = jnp.where(winner, pos, TRASH)

    pk_b = [_bcast(dstp, kk) for kk in range(K)]
    for g in range(ROWS_A // L):
        d16 = base + g * L + lanes
        for kk in range(K):
            d16 = jnp.where(d16 == pk_b[kk], TRASH, d16)
        didx[pl.ds(g * L, L)] = d16

    cp_w.wait()

    def max_body(i, m):
        return jnp.maximum(m, wbuf[pl.ds(i * L, L)])
    m16 = plsc.parallel_loop(0, M // L, unroll=8,
                             carry=jnp.full((L,), -jnp.inf, jnp.float32))(
                                 max_body)
    gmax = _lane_reduce(m16, jnp.maximum)

    def sum_body(i, a):
        return a + jnp.exp(wbuf[pl.ds(i * L, L)] - gmax)
    s16 = plsc.parallel_loop(0, M // L, unroll=8,
                             carry=jnp.zeros((L,), jnp.float32))(sum_body)
    inv = 1.0 / _lane_reduce(s16, jnp.add)

    @plsc.parallel_loop(0, RPW // L, unroll=4)
    def _coef(i):
        w16 = wbuf[pl.ds(q * RPW + i * L, L)]
        cbuf[pl.ds(i * L, L)] = jnp.exp(w16 - gmax) * inv

    cp_sr.wait()
    cp_out = pltpu.async_copy(srows, out.at[didx], sem_o)

    def make_row_body(t, buf):
        def row_body(r, acc8):
            g = t * CHUNK + (r & ~(L - 1))
            c16 = cbuf[pl.ds(g, L)]
            crb = _bcast(c16, r & (L - 1))
            accs = list(acc8)
            for j in range(DL):
                accs[j] = accs[j] + crb * buf[r, pl.ds(j * L, L)]
            return tuple(accs)
        return row_body

    acc8 = tuple(jnp.zeros((L,), jnp.float32) for _ in range(DL))
    for t in range(NCHUNK):
        b = t % NBUF
        handles[b].wait()
        acc8 = plsc.parallel_loop(0, CHUNK, unroll=4, carry=acc8)(
            make_row_body(t, bufs[b]))
        if t + NBUF < NCHUNK:
            handles[b] = pltpu.async_copy(
                table.at[midx_all.at[pl.ds((t + NBUF) * CHUNK, CHUNK)]],
                bufs[b], sems[b])
    cp_out.wait()

    for j in range(DL):
        srows[0, pl.ds(j * L, L)] = acc8[j]
    pltpu.sync_copy(srows.at[0], shared.at[s])
    plsc.subcore_barrier()

    @pl.when(s == 0)
    def _():
        pltpu.sync_copy(shared, comb)
        for kl in range(KPC):
            for j in range(DL):
                v = (comb[kl * WPK + 0, pl.ds(j * L, L)]
                     + comb[kl * WPK + 1, pl.ds(j * L, L)]
                     + comb[kl * WPK + 2, pl.ds(j * L, L)]
                     + comb[kl * WPK + 3, pl.ds(j * L, L)])
                crow[kl, pl.ds(j * L, L)] = v
        zero = jnp.zeros((L,), jnp.float32)
        for kl in range(KPC, L):
            for j in range(DL):
                crow[kl, pl.ds(j * L, L)] = zero
        sel = jnp.where(lanes < KPC, lanes + c * KPC, 0)
        sidx16 = jnp.where(
            lanes < KPC,
            jnp.take_along_axis(dstp, sel, axis=0,
                                mode="promise_in_bounds"),
            TRASH)
        didx16[...] = sidx16
        pltpu.async_copy(crow, out.at[didx16], sem_o).wait()


def kernel(src, tgt, src_lengths, positions, masks, weights, embed_weights):
    out = _sc_combiner(
        embed_weights.astype(jnp.float32),
        src.reshape(S).astype(jnp.int32),
        positions.astype(jnp.int32),
        masks.astype(jnp.int32),
        weights.astype(jnp.float32),
    )
    return (out[:S], tgt, src_lengths)


# speedup vs baseline: 1.0317x; 1.0317x over previous
import functools

import jax
import jax.numpy as jnp
from jax import lax
from jax.experimental import pallas as pl
from jax.experimental.pallas import tpu as pltpu
from jax.experimental.pallas import tpu_sc as plsc

V, D, S, K, M = 100000, 128, 2048, 8, 8192
NC, NS, L = 2, 16, 16
NW = NC * NS
ROWS_A = S // NW
WPK = NW // K
KPC = K // NC
RPW = M // WPK
CHUNK = 128
NBUF = 5
NCHUNK = RPW // CHUNK
GPC = CHUNK // L
DL = D // L
TRASH = S

_mesh = plsc.VectorSubcoreMesh(core_axis_name="c", subcore_axis_name="s")


def _bcast(x, lane):
    idx = jnp.broadcast_to(jnp.asarray(lane, jnp.int32), (L,))
    return jnp.take_along_axis(x, idx, axis=0, mode="promise_in_bounds")


def _lane_reduce(x, op):
    lanes = lax.iota(jnp.int32, L)
    for sh in (8, 4, 2, 1):
        idx = jnp.bitwise_xor(lanes, sh)
        x = op(x, jnp.take_along_axis(x, idx, axis=0,
                                      mode="promise_in_bounds"))
    return x


@functools.partial(
    pl.kernel,
    out_type=jax.ShapeDtypeStruct((S + 8, D), jnp.float32),
    mesh=_mesh,
    scratch_types=[
        pltpu.VMEM((ROWS_A,), jnp.int32),
        pltpu.VMEM((ROWS_A,), jnp.int32),
        pltpu.VMEM((ROWS_A, D), jnp.float32),
        pltpu.VMEM((M,), jnp.float32),
        pltpu.VMEM((RPW,), jnp.int32),
        pltpu.VMEM((RPW,), jnp.float32),
        pltpu.VMEM((L,), jnp.int32),
        pltpu.VMEM((CHUNK, D), jnp.float32),
        pltpu.VMEM((CHUNK, D), jnp.float32),
        pltpu.VMEM((CHUNK, D), jnp.float32),
        pltpu.VMEM((CHUNK, D), jnp.float32),
        pltpu.VMEM((CHUNK, D), jnp.float32),
        pltpu.VMEM((NS, D), jnp.float32),
        pltpu.VMEM((L, D), jnp.float32),
        pltpu.VMEM((L,), jnp.int32),
        pltpu.VMEM_SHARED((NS, D), jnp.float32),
        pltpu.SemaphoreType.DMA,
        pltpu.SemaphoreType.DMA,
        pltpu.SemaphoreType.DMA,
        pltpu.SemaphoreType.DMA,
        pltpu.SemaphoreType.DMA,
        pltpu.SemaphoreType.DMA,
        pltpu.SemaphoreType.DMA,
        pltpu.SemaphoreType.DMA,
        pltpu.SemaphoreType.DMA,
        pltpu.SemaphoreType.DMA,
        pltpu.SemaphoreType.DMA,
    ],
)
def _sc_combiner(table, src_idx, pos16, masks, weights,
                 out, sidx, didx, srows, wbuf, midx_all, cbuf, posv,
                 buf0, buf1, buf2, buf3, buf4, comb, crow, didx16, shared,
                 sem_w, sem_m, sem_s, sem_p, sem_g, sem_o,
                 sem_b0, sem_b1, sem_b2, sem_b3, sem_b4):
    c = lax.axis_index("c")
    s = lax.axis_index("s")
    wid = c * NS + s
    k = c * KPC + s // WPK
    q = s % WPK
    base = wid * ROWS_A

    cp_w = pltpu.async_copy(weights.at[k], wbuf, sem_w)
    cp_m = pltpu.async_copy(masks.at[k, pl.ds(q * RPW, RPW)], midx_all, sem_m)
    cp_si = pltpu.async_copy(src_idx.at[pl.ds(base, ROWS_A)], sidx, sem_s)
    cp_p = pltpu.async_copy(pos16, posv.at[pl.ds(0, K)], sem_p)

    cp_si.wait()
    cp_sr = pltpu.async_copy(table.at[sidx], srows, sem_g)

    cp_m.wait()
    bufs = [buf0, buf1, buf2, buf3, buf4]
    sems = [sem_b0, sem_b1, sem_b2, sem_b3, sem_b4]
    handles = [
        pltpu.async_copy(table.at[midx_all.at[pl.ds(t * CHUNK, CHUNK)]],
                         bufs[t], sems[t])
        for t in range(NBUF)
    ]

    cp_p.wait()
    lanes = lax.iota(jnp.int32, L)
    pos = jnp.where(lanes < K, posv[...], TRASH)
    nxt = jnp.take_along_axis(pos, jnp.minimum(lanes + 1, L - 1), axis=0,
                              mode="promise_in_bounds")
    winner = (lanes < K) & ((pos != nxt) | (lanes == K - 1))
    dstp = jnp.where(winner, pos, TRASH)

    pk_b = [_bcast(dstp, kk) for kk in range(K)]
    for g in range(ROWS_A // L):
        d16 = base + g * L + lanes
        for kk in range(K):
            d16 = jnp.where(d16 == pk_b[kk], TRASH, d16)
        didx[pl.ds(g * L, L)] = d16

    cp_w.wait()

    def max_body(i, m):
        return jnp.maximum(m, wbuf[pl.ds(i * L, L)])
    m16 = plsc.parallel_loop(0, M // L, unroll=8,
                             carry=jnp.full((L,), -jnp.inf, jnp.float32))(
                                 max_body)
    gmax = _lane_reduce(m16, jnp.maximum)

    def sum_body(i, a):
        return a + jnp.exp(wbuf[pl.ds(i * L, L)] - gmax)
    s16 = plsc.parallel_loop(0, M // L, unroll=8,
                             carry=jnp.zeros((L,), jnp.float32))(sum_body)
    inv = 1.0 / _lane_reduce(s16, jnp.add)

    @plsc.parallel_loop(0, RPW // L, unroll=4)
    def _coef(i):
        w16 = wbuf[pl.ds(q * RPW + i * L, L)]
        cbuf[pl.ds(i * L, L)] = jnp.exp(w16 - gmax) * inv

    cp_sr.wait()
    cp_out = pltpu.async_copy(srows, out.at[didx], sem_o)

    def make_row_body(t, buf):
        def row_body(r, acc8):
            g = t * CHUNK + (r & ~(L - 1))
            c16 = cbuf[pl.ds(g, L)]
            crb = _bcast(c16, r & (L - 1))
            accs = list(acc8)
            for j in range(DL):
                accs[j] = accs[j] + crb * buf[r, pl.ds(j * L, L)]
            return tuple(accs)
        return row_body

    acc8 = tuple(jnp.zeros((L,), jnp.float32) for _ in range(DL))
    for t in range(NCHUNK):
        b = t % NBUF
        handles[b].wait()
        acc8 = plsc.parallel_loop(0, CHUNK, unroll=4, carry=acc8)(
            make_row_body(t, bufs[b]))
        if t + NBUF < NCHUNK:
            handles[b] = pltpu.async_copy(
                table.at[midx_all.at[pl.ds((t + NBUF) * CHUNK, CHUNK)]],
                bufs[b], sems[b])
    cp_out.wait()

    for j in range(DL):
        srows[0, pl.ds(j * L, L)] = acc8[j]
    pltpu.sync_copy(srows.at[0], shared.at[s])
    plsc.subcore_barrier()

    @pl.when(s == 0)
    def _():
        pltpu.sync_copy(shared, comb)
        for kl in range(KPC):
            for j in range(DL):
                v = (comb[kl * WPK + 0, pl.ds(j * L, L)]
                     + comb[kl * WPK + 1, pl.ds(j * L, L)]
                     + comb[kl * WPK + 2, pl.ds(j * L, L)]
                     + comb[kl * WPK + 3, pl.ds(j * L, L)])
                crow[kl, pl.ds(j * L, L)] = v
        zero = jnp.zeros((L,), jnp.float32)
        for kl in range(KPC, L):
            for j in range(DL):
                crow[kl, pl.ds(j * L, L)] = zero
        sel = jnp.where(lanes < KPC, lanes + c * KPC, 0)
        sidx16 = jnp.where(
            lanes < KPC,
            jnp.take_along_axis(dstp, sel, axis=0,
                                mode="promise_in_bounds"),
            TRASH)
        didx16[...] = sidx16
        pltpu.async_copy(crow, out.at[didx16], sem_o).wait()


def kernel(src, tgt, src_lengths, positions, masks, weights, embed_weights):
    out = _sc_combiner(
        embed_weights.astype(jnp.float32),
        src.reshape(S).astype(jnp.int32),
        positions.astype(jnp.int32),
        masks.astype(jnp.int32),
        weights.astype(jnp.float32),
    )
    return (out[:S], tgt, src_lengths)
